# Initial kernel scaffold; baseline (speedup 1.0000x reference)
import jax
import jax.numpy as jnp
from jax.experimental import pallas as pl


def kernel(x, edge_index, batch, W1_0, b1_0, W2_0, b2_0, W1_1, b1_1, W2_1, b2_1, W1_2, b1_2, W2_2, b2_2, Wh1, bh1, Wh2, bh2):
    raise NotImplementedError("write your pallas kernel here")


# speedup vs baseline: 4.5714x; 4.5714x over previous
import functools

import jax
import jax.numpy as jnp
from jax import lax
from jax.experimental import pallas as pl
from jax.experimental.pallas import tpu as pltpu
from jax.experimental.pallas import tpu_sc as plsc

N = 10000
E = 320000
D = 128
G = 64

NC = 2
NS = 16
NW = NC * NS
K = 128
CW = 79
EP = NW * CW * K
EPW = CW * K
NA = N + 16
RPT = 624
TAIL = N - NS * RPT


def _agg_body(h_hbm, srcw_hbm, dstw_hbm, zeros_hbm, out_hbm,
              agg_sh, sidx, didx, rows, gsem, isem,
              sidx2, didx2, rows2, gsem2, isem2):
  c = lax.axis_index("c")
  s = lax.axis_index("s")
  w = s * NC + c

  r0 = pl.multiple_of(s * RPT, 8)

  @pl.when(c == 0)
  def _():
    pltpu.sync_copy(h_hbm.at[pl.ds(r0, RPT)], agg_sh.at[pl.ds(r0, RPT)])

    @pl.when(s == 0)
    def _():
      pltpu.sync_copy(h_hbm.at[pl.ds(NS * RPT, TAIL)],
                      agg_sh.at[pl.ds(NS * RPT, TAIL)])

  @pl.when(c != 0)
  def _():
    pltpu.sync_copy(zeros_hbm.at[pl.ds(r0, RPT)], agg_sh.at[pl.ds(r0, RPT)])

    @pl.when(s == 0)
    def _():
      pltpu.sync_copy(zeros_hbm.at[pl.ds(NS * RPT, TAIL)],
                      agg_sh.at[pl.ds(NS * RPT, TAIL)])

  plsc.subcore_barrier()

  bufs = ((sidx, didx, rows, gsem, isem), (sidx2, didx2, rows2, gsem2, isem2))

  def fire_idx(j, b):
    off = pl.multiple_of(j * K, K)
    sb, db, _, _, isem_b = bufs[b]
    pltpu.async_copy(srcw_hbm.at[w, pl.ds(off, K)], sb, isem_b)
    pltpu.async_copy(dstw_hbm.at[w, pl.ds(off, K)], db, isem_b)

  def wait_idx(b):
    sb, db, _, _, isem_b = bufs[b]
    pltpu.make_async_copy(srcw_hbm.at[w, pl.ds(0, K)], sb, isem_b).wait()
    pltpu.make_async_copy(dstw_hbm.at[w, pl.ds(0, K)], db, isem_b).wait()

  def fire_gather(b):
    sb, _, rb, gsem_b, _ = bufs[b]
    pltpu.async_copy(h_hbm.at[sb], rb, gsem_b)

  def wait_gather_scatter(b):
    sb, db, rb, gsem_b, _ = bufs[b]
    pltpu.make_async_copy(h_hbm.at[sb], rb, gsem_b).wait()
    pltpu.sync_copy(rb, agg_sh.at[db], add=True)

  fire_idx(0, 0)
  wait_idx(0)
  fire_gather(0)
  fire_idx(1, 1)

  def pair(i, carry):
    for b in range(2):
      j = i * 2 + b
      wait_idx(1 - b)
      fire_gather(1 - b)
      wait_gather_scatter(b)

      @pl.when(j + 2 < CW)
      def _():
        fire_idx(j + 2, b)
    return carry

  lax.fori_loop(0, CW // 2, pair, 0, unroll=False)
  if CW % 2:
    wait_gather_scatter(0)

  plsc.subcore_barrier()

  pltpu.sync_copy(agg_sh.at[pl.ds(r0, RPT)], out_hbm.at[c, pl.ds(r0, RPT)])

  @pl.when(s == 0)
  def _():
    pltpu.sync_copy(agg_sh.at[pl.ds(NS * RPT, TAIL)],
                    out_hbm.at[c, pl.ds(NS * RPT, TAIL)])


def _make_agg():
  mesh = plsc.VectorSubcoreMesh(core_axis_name="c", subcore_axis_name="s")
  return pl.kernel(
      _agg_body,
      out_type=jax.ShapeDtypeStruct((NC, N, D), jnp.float32),
      mesh=mesh,
      scratch_types=[
          pltpu.VMEM_SHARED((NA, D), jnp.float32),
          pltpu.VMEM((K,), jnp.int32),
          pltpu.VMEM((K,), jnp.int32),
          pltpu.VMEM((K, D), jnp.float32),
          pltpu.SemaphoreType.DMA,
          pltpu.SemaphoreType.DMA,
          pltpu.VMEM((K,), jnp.int32),
          pltpu.VMEM((K,), jnp.int32),
          pltpu.VMEM((K, D), jnp.float32),
          pltpu.SemaphoreType.DMA,
          pltpu.SemaphoreType.DMA,
      ],
  )


R = 1000


def _mlp_body(p0, p1, w1, b1, w2, b2, o):
  a = p0[0] + p1[0]
  t = jnp.dot(a, w1[...], preferred_element_type=jnp.float32) + b1[...]
  t = jnp.maximum(t, 0.0)
  u = jnp.dot(t, w2[...], preferred_element_type=jnp.float32) + b2[...]
  o[...] = jnp.maximum(u, 0.0)


def _mlp3_body(p0, p1, w1, b1, w2, b2, batch, o, pool, cnt):
  a = p0[0] + p1[0]
  t = jnp.dot(a, w1[...], preferred_element_type=jnp.float32) + b1[...]
  t = jnp.maximum(t, 0.0)
  u = jnp.dot(t, w2[...], preferred_element_type=jnp.float32) + b2[...]
  h = jnp.maximum(u, 0.0)
  o[...] = h
  m = (batch[...] == lax.broadcasted_iota(jnp.int32, (1, G), 1)).astype(
      jnp.float32)
  ps = lax.dot_general(m, h, (((0,), (0,)), ((), ())),
                       preferred_element_type=jnp.float32)

  @pl.when(pl.program_id(0) == 0)
  def _():
    pool[...] = jnp.zeros_like(pool)
    cnt[...] = jnp.zeros_like(cnt)

  pool[...] += ps
  cnt[...] += jnp.sum(m, axis=0, keepdims=True)


def _head_body(pool, cnt, wh1, bh1, wh2, bh2, o):
  c = jnp.maximum(cnt[...], 1.0)
  mean = pool[...] / c.reshape(G, 1)
  t = jnp.maximum(
      jnp.dot(mean, wh1[...], preferred_element_type=jnp.float32) + bh1[...],
      0.0)
  o[...] = jnp.dot(t, wh2[...], preferred_element_type=jnp.float32) + bh2[...]


def _mlp_call(p, w1, b1, w2, b2):
  grid = N // R
  return pl.pallas_call(
      _mlp_body,
      grid=(grid,),
      in_specs=[
          pl.BlockSpec((1, R, D), lambda i: (0, i, 0)),
          pl.BlockSpec((1, R, D), lambda i: (1, i, 0)),
          pl.BlockSpec((D, D), lambda i: (0, 0)),
          pl.BlockSpec((1, D), lambda i: (0, 0)),
          pl.BlockSpec((D, D), lambda i: (0, 0)),
          pl.BlockSpec((1, D), lambda i: (0, 0)),
      ],
      out_specs=pl.BlockSpec((R, D), lambda i: (i, 0)),
      out_shape=jax.ShapeDtypeStruct((N, D), jnp.float32),
  )(p, p, w1, b1.reshape(1, D), w2, b2.reshape(1, D))


def _mlp3_call(p, w1, b1, w2, b2, batch2d):
  grid = N // R
  return pl.pallas_call(
      _mlp3_body,
      grid=(grid,),
      in_specs=[
          pl.BlockSpec((1, R, D), lambda i: (0, i, 0)),
          pl.BlockSpec((1, R, D), lambda i: (1, i, 0)),
          pl.BlockSpec((D, D), lambda i: (0, 0)),
          pl.BlockSpec((1, D), lambda i: (0, 0)),
          pl.BlockSpec((D, D), lambda i: (0, 0)),
          pl.BlockSpec((1, D), lambda i: (0, 0)),
          pl.BlockSpec((R, 1), lambda i: (i, 0)),
      ],
      out_specs=[
          pl.BlockSpec((R, D), lambda i: (i, 0)),
          pl.BlockSpec((G, D), lambda i: (0, 0)),
          pl.BlockSpec((1, G), lambda i: (0, 0)),
      ],
      out_shape=[
          jax.ShapeDtypeStruct((N, D), jnp.float32),
          jax.ShapeDtypeStruct((G, D), jnp.float32),
          jax.ShapeDtypeStruct((1, G), jnp.float32),
      ],
  )(p, p, w1, b1.reshape(1, D), w2, b2.reshape(1, D), batch2d)


def _head_call(pool, cnt, wh1, bh1, wh2, bh2):
  return pl.pallas_call(
      _head_body,
      out_shape=jax.ShapeDtypeStruct((G, 1), jnp.float32),
  )(pool, cnt, wh1, bh1.reshape(1, G), wh2, bh2.reshape(1, 1))


def kernel(x, edge_index, batch, W1_0, b1_0, W2_0, b2_0, W1_1, b1_1, W2_1,
           b2_1, W1_2, b1_2, W2_2, b2_2, Wh1, bh1, Wh2, bh2):
  src = edge_index[0].astype(jnp.int32)
  dst = edge_index[1].astype(jnp.int32)
  npad = EP - E
  srcw = jnp.concatenate([src, jnp.zeros((npad,), jnp.int32)]).reshape(NW, EPW)
  dstw = jnp.concatenate([dst, jnp.full((npad,), N, jnp.int32)]).reshape(
      NW, EPW)
  zeros = jnp.zeros((N, D), jnp.float32)
  batch2d = batch.astype(jnp.int32).reshape(N, 1)

  agg = _make_agg()

  p = agg(x, srcw, dstw, zeros)
  h = _mlp_call(p, W1_0, b1_0, W2_0, b2_0)
  p = agg(h, srcw, dstw, zeros)
  h = _mlp_call(p, W1_1, b1_1, W2_1, b2_1)
  p = agg(h, srcw, dstw, zeros)
  h, pool, cnt = _mlp3_call(p, W1_2, b1_2, W2_2, b2_2, batch2d)
  wh1h = Wh1.reshape(D, G)
  return _head_call(pool, cnt, wh1h, bh1, Wh2, bh2)


# speedup vs baseline: 10.4301x; 2.2816x over previous
import functools

import jax
import jax.numpy as jnp
from jax import lax
from jax.experimental import pallas as pl
from jax.experimental.pallas import tpu as pltpu
from jax.experimental.pallas import tpu_sc as plsc

N = 10000
E = 320000
D = 128
G = 64

NC = 2
NS = 16
NW = NC * NS
K = 128
CW = 79
EP = NW * CW * K
EPW = CW * K
NA = N + 16
RPT = 624
TAIL = N - NS * RPT


def _agg_body(h_hbm, srcw_hbm, dstw_hbm, zeros_hbm, out_hbm,
              agg_sh, sidx, didx, rows, gsem, isem,
              sidx2, didx2, rows2, gsem2, isem2):
  c = lax.axis_index("c")
  s = lax.axis_index("s")
  w = s * NC + c

  r0 = pl.multiple_of(s * RPT, 8)

  @pl.when(c == 0)
  def _():
    pltpu.sync_copy(h_hbm.at[pl.ds(r0, RPT)], agg_sh.at[pl.ds(r0, RPT)])

    @pl.when(s == 0)
    def _():
      pltpu.sync_copy(h_hbm.at[pl.ds(NS * RPT, TAIL)],
                      agg_sh.at[pl.ds(NS * RPT, TAIL)])

  @pl.when(c != 0)
  def _():
    pltpu.sync_copy(zeros_hbm.at[pl.ds(r0, RPT)], agg_sh.at[pl.ds(r0, RPT)])

    @pl.when(s == 0)
    def _():
      pltpu.sync_copy(zeros_hbm.at[pl.ds(NS * RPT, TAIL)],
                      agg_sh.at[pl.ds(NS * RPT, TAIL)])

  plsc.subcore_barrier()

  bufs = ((sidx, didx, rows, gsem, isem), (sidx2, didx2, rows2, gsem2, isem2))

  def fire_idx(j, b):
    off = pl.multiple_of(j * K, K)
    sb, db, _, _, isem_b = bufs[b]
    pltpu.async_copy(srcw_hbm.at[w, pl.ds(off, K)], sb, isem_b)
    pltpu.async_copy(dstw_hbm.at[w, pl.ds(off, K)], db, isem_b)

  def wait_idx(b):
    sb, db, _, _, isem_b = bufs[b]
    pltpu.make_async_copy(srcw_hbm.at[w, pl.ds(0, K)], sb, isem_b).wait()
    pltpu.make_async_copy(dstw_hbm.at[w, pl.ds(0, K)], db, isem_b).wait()

  def fire_gather(b):
    sb, _, rb, gsem_b, _ = bufs[b]
    pltpu.async_copy(h_hbm.at[sb], rb, gsem_b)

  def wait_gather_scatter(b):
    sb, db, rb, gsem_b, _ = bufs[b]
    pltpu.make_async_copy(h_hbm.at[sb], rb, gsem_b).wait()
    pltpu.sync_copy(rb, agg_sh.at[db], add=True)

  fire_idx(0, 0)
  wait_idx(0)
  fire_gather(0)
  fire_idx(1, 1)

  def pair(i, carry):
    for b in range(2):
      j = i * 2 + b
      wait_idx(1 - b)
      fire_gather(1 - b)
      wait_gather_scatter(b)

      @pl.when(j + 2 < CW)
      def _():
        fire_idx(j + 2, b)
    return carry

  lax.fori_loop(0, CW // 2, pair, 0, unroll=False)
  if CW % 2:
    wait_gather_scatter(0)

  plsc.subcore_barrier()

  pltpu.sync_copy(agg_sh.at[pl.ds(r0, RPT)], out_hbm.at[c, pl.ds(r0, RPT)])

  @pl.when(s == 0)
  def _():
    pltpu.sync_copy(agg_sh.at[pl.ds(NS * RPT, TAIL)],
                    out_hbm.at[c, pl.ds(NS * RPT, TAIL)])


def _make_agg():
  mesh = plsc.VectorSubcoreMesh(core_axis_name="c", subcore_axis_name="s")
  return pl.kernel(
      _agg_body,
      out_type=jax.ShapeDtypeStruct((NC, N, D), jnp.float32),
      mesh=mesh,
      scratch_types=[
          pltpu.VMEM_SHARED((NA, D), jnp.float32),
          pltpu.VMEM((K,), jnp.int32),
          pltpu.VMEM((K,), jnp.int32),
          pltpu.VMEM((K, D), jnp.float32),
          pltpu.SemaphoreType.DMA,
          pltpu.SemaphoreType.DMA,
          pltpu.VMEM((K,), jnp.int32),
          pltpu.VMEM((K,), jnp.int32),
          pltpu.VMEM((K, D), jnp.float32),
          pltpu.SemaphoreType.DMA,
          pltpu.SemaphoreType.DMA,
      ],
  )


R = 1000


def _mlp_body(p0, p1, w1, b1, w2, b2, o):
  a = p0[0] + p1[0]
  t = jnp.dot(a, w1[...], preferred_element_type=jnp.float32) + b1[...]
  t = jnp.maximum(t, 0.0)
  u = jnp.dot(t, w2[...], preferred_element_type=jnp.float32) + b2[...]
  o[...] = jnp.maximum(u, 0.0)


def _mlp3_body(p0, p1, w1, b1, w2, b2, batch, o, pool, cnt):
  a = p0[0] + p1[0]
  t = jnp.dot(a, w1[...], preferred_element_type=jnp.float32) + b1[...]
  t = jnp.maximum(t, 0.0)
  u = jnp.dot(t, w2[...], preferred_element_type=jnp.float32) + b2[...]
  h = jnp.maximum(u, 0.0)
  o[...] = h
  m = (batch[...] == lax.broadcasted_iota(jnp.int32, (1, G), 1)).astype(
      jnp.float32)
  ps = lax.dot_general(m, h, (((0,), (0,)), ((), ())),
                       preferred_element_type=jnp.float32)

  @pl.when(pl.program_id(0) == 0)
  def _():
    pool[...] = jnp.zeros_like(pool)
    cnt[...] = jnp.zeros_like(cnt)

  pool[...] += ps
  cnt[...] += jnp.sum(m, axis=0, keepdims=True)


def _head_body(pool, cnt, wh1, bh1, wh2, bh2, o):
  c = jnp.maximum(cnt[...], 1.0)
  mean = pool[...] / c.reshape(G, 1)
  t = jnp.maximum(
      jnp.dot(mean, wh1[...], preferred_element_type=jnp.float32) + bh1[...],
      0.0)
  o[...] = jnp.dot(t, wh2[...], preferred_element_type=jnp.float32) + bh2[...]


def _mlp_call(p, w1, b1, w2, b2):
  grid = N // R
  return pl.pallas_call(
      _mlp_body,
      grid=(grid,),
      in_specs=[
          pl.BlockSpec((1, R, D), lambda i: (0, i, 0)),
          pl.BlockSpec((1, R, D), lambda i: (1, i, 0)),
          pl.BlockSpec((D, D), lambda i: (0, 0)),
          pl.BlockSpec((1, D), lambda i: (0, 0)),
          pl.BlockSpec((D, D), lambda i: (0, 0)),
          pl.BlockSpec((1, D), lambda i: (0, 0)),
      ],
      out_specs=pl.BlockSpec((R, D), lambda i: (i, 0)),
      out_shape=jax.ShapeDtypeStruct((N, D), jnp.float32),
  )(p, p, w1, b1.reshape(1, D), w2, b2.reshape(1, D))


def _mlp3_call(p, w1, b1, w2, b2, batch2d):
  grid = N // R
  return pl.pallas_call(
      _mlp3_body,
      grid=(grid,),
      in_specs=[
          pl.BlockSpec((1, R, D), lambda i: (0, i, 0)),
          pl.BlockSpec((1, R, D), lambda i: (1, i, 0)),
          pl.BlockSpec((D, D), lambda i: (0, 0)),
          pl.BlockSpec((1, D), lambda i: (0, 0)),
          pl.BlockSpec((D, D), lambda i: (0, 0)),
          pl.BlockSpec((1, D), lambda i: (0, 0)),
          pl.BlockSpec((R, 1), lambda i: (i, 0)),
      ],
      out_specs=[
          pl.BlockSpec((R, D), lambda i: (i, 0)),
          pl.BlockSpec((G, D), lambda i: (0, 0)),
          pl.BlockSpec((1, G), lambda i: (0, 0)),
      ],
      out_shape=[
          jax.ShapeDtypeStruct((N, D), jnp.float32),
          jax.ShapeDtypeStruct((G, D), jnp.float32),
          jax.ShapeDtypeStruct((1, G), jnp.float32),
      ],
  )(p, p, w1, b1.reshape(1, D), w2, b2.reshape(1, D), batch2d)


def _head_call(pool, cnt, wh1, bh1, wh2, bh2):
  return pl.pallas_call(
      _head_body,
      out_shape=jax.ShapeDtypeStruct((G, 1), jnp.float32),
  )(pool, cnt, wh1, bh1.reshape(1, G), wh2, bh2.reshape(1, 1))


def kernel(x, edge_index, batch, W1_0, b1_0, W2_0, b2_0, W1_1, b1_1, W2_1,
           b2_1, W1_2, b1_2, W2_2, b2_2, Wh1, bh1, Wh2, bh2):
  src = edge_index[0].astype(jnp.int32)
  dst = edge_index[1].astype(jnp.int32)
  npad = EP - E
  pad_i = jnp.arange(npad, dtype=jnp.int32)
  srcw = jnp.concatenate([src, pad_i % N]).reshape(NW, EPW)
  dstw = jnp.concatenate([dst, N + (pad_i % 16)]).reshape(NW, EPW)
  zeros = jnp.zeros((N, D), jnp.float32)
  batch2d = batch.astype(jnp.int32).reshape(N, 1)

  agg = _make_agg()

  p = agg(x, srcw, dstw, zeros)
  h = _mlp_call(p, W1_0, b1_0, W2_0, b2_0)
  p = agg(h, srcw, dstw, zeros)
  h = _mlp_call(p, W1_1, b1_1, W2_1, b2_1)
  p = agg(h, srcw, dstw, zeros)
  h, pool, cnt = _mlp3_call(p, W1_2, b1_2, W2_2, b2_2, batch2d)
  wh1h = Wh1.reshape(D, G)
  return _head_call(pool, cnt, wh1h, bh1, Wh2, bh2)


# speedup vs baseline: 10.7998x; 1.0354x over previous
import functools

import jax
import jax.numpy as jnp
from jax import lax
from jax.experimental import pallas as pl
from jax.experimental.pallas import tpu as pltpu
from jax.experimental.pallas import tpu_sc as plsc

N = 10000
E = 320000
D = 128
G = 64

NC = 2
NS = 16
NW = NC * NS
K = 128
CW = 79
EP = NW * CW * K
EPW = CW * K
NA = N + 16
RPT = 624
TAIL = N - NS * RPT


def _agg_body(h_hbm, srcw_hbm, dstw_hbm, zeros_hbm, out_hbm,
              agg_sh, sidx, didx, rows, gsem, isem,
              sidx2, didx2, rows2, gsem2, isem2,
              sidx3, didx3, rows3, gsem3, isem3):
  c = lax.axis_index("c")
  s = lax.axis_index("s")
  w = s * NC + c

  r0 = pl.multiple_of(s * RPT, 8)

  @pl.when(c == 0)
  def _():
    pltpu.sync_copy(h_hbm.at[pl.ds(r0, RPT)], agg_sh.at[pl.ds(r0, RPT)])

    @pl.when(s == 0)
    def _():
      pltpu.sync_copy(h_hbm.at[pl.ds(NS * RPT, TAIL)],
                      agg_sh.at[pl.ds(NS * RPT, TAIL)])

  @pl.when(c != 0)
  def _():
    pltpu.sync_copy(zeros_hbm.at[pl.ds(r0, RPT)], agg_sh.at[pl.ds(r0, RPT)])

    @pl.when(s == 0)
    def _():
      pltpu.sync_copy(zeros_hbm.at[pl.ds(NS * RPT, TAIL)],
                      agg_sh.at[pl.ds(NS * RPT, TAIL)])

  plsc.subcore_barrier()

  bufs = ((sidx, didx, rows, gsem, isem),
          (sidx2, didx2, rows2, gsem2, isem2),
          (sidx3, didx3, rows3, gsem3, isem3))

  def fire_idx(j, b):
    off = pl.multiple_of(j * K, K)
    sb, db, _, _, isem_b = bufs[b]
    pltpu.async_copy(srcw_hbm.at[w, pl.ds(off, K)], sb, isem_b)
    pltpu.async_copy(dstw_hbm.at[w, pl.ds(off, K)], db, isem_b)

  def wait_idx(b):
    sb, db, _, _, isem_b = bufs[b]
    pltpu.make_async_copy(srcw_hbm.at[w, pl.ds(0, K)], sb, isem_b).wait()
    pltpu.make_async_copy(dstw_hbm.at[w, pl.ds(0, K)], db, isem_b).wait()

  def fire_gather(b):
    sb, _, rb, gsem_b, _ = bufs[b]
    pltpu.async_copy(h_hbm.at[sb], rb, gsem_b)

  def wait_gather_scatter(b):
    sb, db, rb, gsem_b, _ = bufs[b]
    pltpu.make_async_copy(h_hbm.at[sb], rb, gsem_b).wait()
    pltpu.sync_copy(rb, agg_sh.at[db], add=True)

  fire_idx(0, 0)
  fire_idx(1, 1)
  fire_idx(2, 2)
  wait_idx(0)
  fire_gather(0)
  wait_idx(1)
  fire_gather(1)

  def triple(i, carry):
    for b in range(3):
      j = i * 3 + b

      @pl.when(j + 2 < CW)
      def _():
        wait_idx((b + 2) % 3)
        fire_gather((b + 2) % 3)

      wait_gather_scatter(b)

      @pl.when(j + 3 < CW)
      def _():
        fire_idx(j + 3, b)
    return carry

  lax.fori_loop(0, CW // 3, triple, 0, unroll=False)
  for j in range(3 * (CW // 3), CW):
    wait_gather_scatter(j % 3)

  plsc.subcore_barrier()

  pltpu.sync_copy(agg_sh.at[pl.ds(r0, RPT)], out_hbm.at[c, pl.ds(r0, RPT)])

  @pl.when(s == 0)
  def _():
    pltpu.sync_copy(agg_sh.at[pl.ds(NS * RPT, TAIL)],
                    out_hbm.at[c, pl.ds(NS * RPT, TAIL)])


def _make_agg():
  mesh = plsc.VectorSubcoreMesh(core_axis_name="c", subcore_axis_name="s")
  return pl.kernel(
      _agg_body,
      out_type=jax.ShapeDtypeStruct((NC, N, D), jnp.float32),
      mesh=mesh,
      scratch_types=[
          pltpu.VMEM_SHARED((NA, D), jnp.float32),
          pltpu.VMEM((K,), jnp.int32),
          pltpu.VMEM((K,), jnp.int32),
          pltpu.VMEM((K, D), jnp.float32),
          pltpu.SemaphoreType.DMA,
          pltpu.SemaphoreType.DMA,
          pltpu.VMEM((K,), jnp.int32),
          pltpu.VMEM((K,), jnp.int32),
          pltpu.VMEM((K, D), jnp.float32),
          pltpu.SemaphoreType.DMA,
          pltpu.SemaphoreType.DMA,
          pltpu.VMEM((K,), jnp.int32),
          pltpu.VMEM((K,), jnp.int32),
          pltpu.VMEM((K, D), jnp.float32),
          pltpu.SemaphoreType.DMA,
          pltpu.SemaphoreType.DMA,
      ],
  )


R = 1000


def _mlp_body(p0, p1, w1, b1, w2, b2, o):
  a = p0[0] + p1[0]
  t = jnp.dot(a, w1[...], preferred_element_type=jnp.float32) + b1[...]
  t = jnp.maximum(t, 0.0)
  u = jnp.dot(t, w2[...], preferred_element_type=jnp.float32) + b2[...]
  o[...] = jnp.maximum(u, 0.0)


def _mlp3_body(p0, p1, w1, b1, w2, b2, batch, o, pool, cnt):
  a = p0[0] + p1[0]
  t = jnp.dot(a, w1[...], preferred_element_type=jnp.float32) + b1[...]
  t = jnp.maximum(t, 0.0)
  u = jnp.dot(t, w2[...], preferred_element_type=jnp.float32) + b2[...]
  h = jnp.maximum(u, 0.0)
  o[...] = h
  m = (batch[...] == lax.broadcasted_iota(jnp.int32, (1, G), 1)).astype(
      jnp.float32)
  ps = lax.dot_general(m, h, (((0,), (0,)), ((), ())),
                       preferred_element_type=jnp.float32,
                       precision=lax.Precision.HIGHEST)

  @pl.when(pl.program_id(0) == 0)
  def _():
    pool[...] = jnp.zeros_like(pool)
    cnt[...] = jnp.zeros_like(cnt)

  pool[...] += ps
  cnt[...] += jnp.sum(m, axis=0, keepdims=True)


def _head_body(pool, cnt, wh1, bh1, wh2, bh2, o):
  c = jnp.maximum(cnt[...], 1.0)
  mean = pool[...] / c.reshape(G, 1)
  t = jnp.maximum(
      jnp.dot(mean, wh1[...], preferred_element_type=jnp.float32) + bh1[...],
      0.0)
  o[...] = jnp.dot(t, wh2[...], preferred_element_type=jnp.float32) + bh2[...]


def _mlp_call(p, w1, b1, w2, b2):
  grid = N // R
  return pl.pallas_call(
      _mlp_body,
      grid=(grid,),
      in_specs=[
          pl.BlockSpec((1, R, D), lambda i: (0, i, 0)),
          pl.BlockSpec((1, R, D), lambda i: (1, i, 0)),
          pl.BlockSpec((D, D), lambda i: (0, 0)),
          pl.BlockSpec((1, D), lambda i: (0, 0)),
          pl.BlockSpec((D, D), lambda i: (0, 0)),
          pl.BlockSpec((1, D), lambda i: (0, 0)),
      ],
      out_specs=pl.BlockSpec((R, D), lambda i: (i, 0)),
      out_shape=jax.ShapeDtypeStruct((N, D), jnp.float32),
  )(p, p, w1, b1.reshape(1, D), w2, b2.reshape(1, D))


def _mlp3_call(p, w1, b1, w2, b2, batch2d):
  grid = N // R
  return pl.pallas_call(
      _mlp3_body,
      grid=(grid,),
      in_specs=[
          pl.BlockSpec((1, R, D), lambda i: (0, i, 0)),
          pl.BlockSpec((1, R, D), lambda i: (1, i, 0)),
          pl.BlockSpec((D, D), lambda i: (0, 0)),
          pl.BlockSpec((1, D), lambda i: (0, 0)),
          pl.BlockSpec((D, D), lambda i: (0, 0)),
          pl.BlockSpec((1, D), lambda i: (0, 0)),
          pl.BlockSpec((R, 1), lambda i: (i, 0)),
      ],
      out_specs=[
          pl.BlockSpec((R, D), lambda i: (i, 0)),
          pl.BlockSpec((G, D), lambda i: (0, 0)),
          pl.BlockSpec((1, G), lambda i: (0, 0)),
      ],
      out_shape=[
          jax.ShapeDtypeStruct((N, D), jnp.float32),
          jax.ShapeDtypeStruct((G, D), jnp.float32),
          jax.ShapeDtypeStruct((1, G), jnp.float32),
      ],
  )(p, p, w1, b1.reshape(1, D), w2, b2.reshape(1, D), batch2d)


def _head_call(pool, cnt, wh1, bh1, wh2, bh2):
  return pl.pallas_call(
      _head_body,
      out_shape=jax.ShapeDtypeStruct((G, 1), jnp.float32),
  )(pool, cnt, wh1, bh1.reshape(1, G), wh2, bh2.reshape(1, 1))


def kernel(x, edge_index, batch, W1_0, b1_0, W2_0, b2_0, W1_1, b1_1, W2_1,
           b2_1, W1_2, b1_2, W2_2, b2_2, Wh1, bh1, Wh2, bh2):
  src = edge_index[0].astype(jnp.int32)
  dst = edge_index[1].astype(jnp.int32)
  npad = EP - E
  pad_i = jnp.arange(npad, dtype=jnp.int32)
  srcw = jnp.concatenate([src, pad_i % N]).reshape(NW, EPW)
  dstw = jnp.concatenate([dst, N + (pad_i % 16)]).reshape(NW, EPW)
  zeros = jnp.zeros((N, D), jnp.float32)
  batch2d = batch.astype(jnp.int32).reshape(N, 1)

  agg = _make_agg()

  p = agg(x, srcw, dstw, zeros)
  h = _mlp_call(p, W1_0, b1_0, W2_0, b2_0)
  p = agg(h, srcw, dstw, zeros)
  h = _mlp_call(p, W1_1, b1_1, W2_1, b2_1)
  p = agg(h, srcw, dstw, zeros)
  h, pool, cnt = _mlp3_call(p, W1_2, b1_2, W2_2, b2_2, batch2d)
  wh1h = Wh1.reshape(D, G)
  return _head_call(pool, cnt, wh1h, bh1, Wh2, bh2)
